# Initial kernel scaffold; baseline (speedup 1.0000x reference)
import jax
import jax.numpy as jnp
from jax.experimental import pallas as pl


def kernel(x, edge_index, edge_attr, weight, w_ih, w_hh, b_ih, b_hh):
    raise NotImplementedError("write your pallas kernel here")


# speedup vs baseline: 3.3807x; 3.3807x over previous
import functools

import jax
import jax.numpy as jnp
from jax import lax
from jax.experimental import pallas as pl
from jax.experimental.pallas import tpu as pltpu
from jax.experimental.pallas import tpu_sc as plsc

N = 10000
H = 128
E = 320000
NC = 2
NS = 16
NW = NC * NS
EW = E // NW
K = 80
NCH = EW // K
NP = 10240
RPT = NP // NS


def _sc_edge_body(m_hbm, src_hbm, dst_hbm, ea_hbm, z_hbm, out_hbm,
                  src_v, dst_v, ea_v, rows_v, agg_sh, sem):
    c = lax.axis_index("c")
    s = lax.axis_index("s")
    wid = s * NC + c

    pltpu.sync_copy(z_hbm, agg_sh.at[pl.ds(s * RPT, RPT)])
    plsc.subcore_barrier()

    base = wid * EW

    def chunk(ci, carry):
        off = base + ci * K
        pltpu.sync_copy(src_hbm.at[pl.ds(off, K)], src_v)
        pltpu.sync_copy(dst_hbm.at[pl.ds(off, K)], dst_v)
        pltpu.sync_copy(ea_hbm.at[pl.ds(off, K)], ea_v.at[pl.ds(0, K)])
        pltpu.async_copy(m_hbm.at[src_v], rows_v, sem).wait()

        def edge(e, _):
            w = ea_v[pl.ds(e, 16)]
            eav = jnp.full((16,), w[0], jnp.float32)
            for j in range(H // 16):
                rows_v[e, pl.ds(16 * j, 16)] = (
                    rows_v[e, pl.ds(16 * j, 16)] * eav)
            return 0

        lax.fori_loop(0, K, edge, 0)

        pltpu.sync_copy(rows_v, agg_sh.at[dst_v], add=True)
        return carry

    lax.fori_loop(0, NCH, chunk, 0)
    plsc.subcore_barrier()

    pltpu.sync_copy(agg_sh.at[pl.ds(s * RPT, RPT)],
                    out_hbm.at[pl.ds(c * NP + s * RPT, RPT)])


@functools.partial(
    pl.kernel,
    out_type=jax.ShapeDtypeStruct((2 * NP, H), jnp.float32),
    mesh=plsc.VectorSubcoreMesh(core_axis_name="c", subcore_axis_name="s"),
    scratch_types=[
        pltpu.VMEM((K,), jnp.int32),
        pltpu.VMEM((K,), jnp.int32),
        pltpu.VMEM((K + 16,), jnp.float32),
        pltpu.VMEM((K, H), jnp.float32),
        pltpu.VMEM_SHARED((NP, H), jnp.float32),
        pltpu.SemaphoreType.DMA,
    ],
)
def _sc_edge(m_hbm, src_hbm, dst_hbm, ea_hbm, z_hbm, out_hbm,
             src_v, dst_v, ea_v, rows_v, agg_sh, sem):
    _sc_edge_body(m_hbm, src_hbm, dst_hbm, ea_hbm, z_hbm, out_hbm,
                  src_v, dst_v, ea_v, rows_v, agg_sh, sem)


BN = 1000


def _mm_body(x_ref, w_ref, o_ref):
    o_ref[...] = jnp.dot(x_ref[...], w_ref[...],
                         preferred_element_type=jnp.float32)


def _tc_matmul(x, w):
    return pl.pallas_call(
        _mm_body,
        grid=(N // BN,),
        in_specs=[pl.BlockSpec((BN, H), lambda i: (i, 0)),
                  pl.BlockSpec((H, H), lambda i: (0, 0))],
        out_specs=pl.BlockSpec((BN, H), lambda i: (i, 0)),
        out_shape=jax.ShapeDtypeStruct((N, H), jnp.float32),
    )(x, w)


def _gru_math(p0, p1, h, wihT, whhT, bih, bhh):
    agg = p0 + p1
    gi = jnp.dot(agg, wihT, preferred_element_type=jnp.float32) + bih
    gh = jnp.dot(h, whhT, preferred_element_type=jnp.float32) + bhh
    r = jax.nn.sigmoid(gi[:, :H] + gh[:, :H])
    z = jax.nn.sigmoid(gi[:, H:2 * H] + gh[:, H:2 * H])
    n = jnp.tanh(gi[:, 2 * H:] + r * gh[:, 2 * H:])
    return (1.0 - z) * n + z * h


def _gru_fused_body(p0_ref, p1_ref, h_ref, wihT_ref, whhT_ref, bih_ref,
                    bhh_ref, wn_ref, oh_ref, om_ref):
    hn = _gru_math(p0_ref[...], p1_ref[...], h_ref[...], wihT_ref[...],
                   whhT_ref[...], bih_ref[...], bhh_ref[...])
    oh_ref[...] = hn
    om_ref[...] = jnp.dot(hn, wn_ref[...], preferred_element_type=jnp.float32)


def _gru_final_body(p0_ref, p1_ref, h_ref, wihT_ref, whhT_ref, bih_ref,
                    bhh_ref, oh_ref):
    oh_ref[...] = _gru_math(p0_ref[...], p1_ref[...], h_ref[...], wihT_ref[...],
                            whhT_ref[...], bih_ref[...], bhh_ref[...])


def _blk(r, c):
    return pl.BlockSpec((r, c), lambda i: (i, 0))


def _full(r, c):
    return pl.BlockSpec((r, c), lambda i: (0, 0))


def _tc_gru_fused(p0, p1, h, wihT, whhT, bih, bhh, wn):
    return pl.pallas_call(
        _gru_fused_body,
        grid=(N // BN,),
        in_specs=[_blk(BN, H), _blk(BN, H), _blk(BN, H),
                  _full(H, 3 * H), _full(H, 3 * H),
                  _full(1, 3 * H), _full(1, 3 * H), _full(H, H)],
        out_specs=[_blk(BN, H), _blk(BN, H)],
        out_shape=[jax.ShapeDtypeStruct((N, H), jnp.float32),
                   jax.ShapeDtypeStruct((N, H), jnp.float32)],
    )(p0, p1, h, wihT, whhT, bih, bhh, wn)


def _tc_gru_final(p0, p1, h, wihT, whhT, bih, bhh):
    return pl.pallas_call(
        _gru_final_body,
        grid=(N // BN,),
        in_specs=[_blk(BN, H), _blk(BN, H), _blk(BN, H),
                  _full(H, 3 * H), _full(H, 3 * H),
                  _full(1, 3 * H), _full(1, 3 * H)],
        out_specs=_blk(BN, H),
        out_shape=jax.ShapeDtypeStruct((N, H), jnp.float32),
    )(p0, p1, h, wihT, whhT, bih, bhh)


def kernel(x, edge_index, edge_attr, weight, w_ih, w_hh, b_ih, b_hh):
    src = edge_index[0].astype(jnp.int32)
    dst = edge_index[1].astype(jnp.int32)
    ea = edge_attr.astype(jnp.float32)
    wihT = w_ih.T
    whhT = w_hh.T
    bih = b_ih.reshape(1, 3 * H)
    bhh = b_hh.reshape(1, 3 * H)
    zeros = jnp.zeros((RPT, H), jnp.float32)

    h = x
    m = _tc_matmul(h, weight[0])
    p = _sc_edge(m, src, dst, ea, zeros)
    h, m = _tc_gru_fused(p[:N], p[NP:NP + N], h, wihT, whhT, bih, bhh,
                         weight[1])
    p = _sc_edge(m, src, dst, ea, zeros)
    h = _tc_gru_final(p[:N], p[NP:NP + N], h, wihT, whhT, bih, bhh)
    return h
